# baseline (speedup 1.0000x reference)
import functools

import jax
import jax.numpy as jnp
from jax import lax
from jax.experimental import pallas as pl
from jax.experimental.pallas import tpu as pltpu
from jax.experimental.pallas import tpu_sc as plsc

_NC = 2
_NS = 16
_BLK = 25000


def _copy_block(ent_ref, ent_out):
    ent_out[...] = ent_ref[...]


def _sc_rel_copy(rel_emb):
    mesh = plsc.VectorSubcoreMesh(core_axis_name="c", subcore_axis_name="s")

    @functools.partial(
        pl.kernel,
        mesh=mesh,
        out_type=jax.ShapeDtypeStruct(rel_emb.shape, rel_emb.dtype),
        scratch_types=[pltpu.SemaphoreType.DMA],
    )
    def _body(rel_in, rel_out, sem):
        wid = lax.axis_index("s") * _NC + lax.axis_index("c")

        @pl.when(wid == 0)
        def _():
            cp = pltpu.make_async_copy(rel_in, rel_out, sem)
            cp.start()
            cp.wait()

    return _body(rel_emb)


def kernel(x_dict, edge_index, entity_emb, rel_emb):
    del x_dict, edge_index
    n_ent, d_ent = entity_emb.shape
    rel = _sc_rel_copy(rel_emb)
    ent = pl.pallas_call(
        _copy_block,
        grid=(n_ent // _BLK,),
        in_specs=[pl.BlockSpec((_BLK, d_ent), lambda i: (i, 0))],
        out_specs=pl.BlockSpec((_BLK, d_ent), lambda i: (i, 0)),
        out_shape=jax.ShapeDtypeStruct((n_ent, d_ent), entity_emb.dtype),
    )(entity_emb)
    return (ent, rel)


# speedup vs baseline: 1.0459x; 1.0459x over previous
import jax
import jax.numpy as jnp
from jax.experimental import pallas as pl
from jax.experimental.pallas import tpu as pltpu

_BLK = 25000


def _copy_tables(ent_ref, rel_ref, ent_out, rel_out):
    ent_out[...] = ent_ref[...]

    @pl.when(pl.program_id(0) == 0)
    def _():
        rel_out[...] = rel_ref[...]


def kernel(x_dict, edge_index, entity_emb, rel_emb):
    del x_dict, edge_index
    n_ent, d_ent = entity_emb.shape
    n_rel, d_rel = rel_emb.shape
    ent, rel = pl.pallas_call(
        _copy_tables,
        grid=(n_ent // _BLK,),
        in_specs=[
            pl.BlockSpec((_BLK, d_ent), lambda i: (i, 0)),
            pl.BlockSpec((n_rel, d_rel), lambda i: (0, 0)),
        ],
        out_specs=[
            pl.BlockSpec((_BLK, d_ent), lambda i: (i, 0)),
            pl.BlockSpec((n_rel, d_rel), lambda i: (0, 0)),
        ],
        out_shape=[
            jax.ShapeDtypeStruct((n_ent, d_ent), entity_emb.dtype),
            jax.ShapeDtypeStruct((n_rel, d_rel), rel_emb.dtype),
        ],
        compiler_params=pltpu.CompilerParams(
            dimension_semantics=("parallel",),
        ),
    )(entity_emb, rel_emb)
    return (ent, rel)
